# baseline (speedup 1.0000x reference)
import jax
import jax.numpy as jnp
from jax import lax
from jax.experimental import pallas as pl
from jax.experimental.pallas import tpu as pltpu
from jax.experimental.pallas import tpu_sc as plsc

B, T, F = 16, 2048, 128
M = 4
LAMBDA_MIN, LAMBDA_MAX = 0.1, 0.4

NC, NS = 2, 16
HALF = T // NC
C = 64
NCHUNK = HALF // C
GRP = F // 16
L = 16


def _sc_body(inp_hbm, maskf_hbm, cache_hbm, meta_hbm, out_hbm,
             meta_v, mask_v, inp_v, win_v, out_v, in_sem, out_sem):
    c = lax.axis_index("c")
    s = lax.axis_index("s")
    b = s
    t_base = c * HALF

    pltpu.sync_copy(meta_hbm.at[0, b], meta_v.at[0])
    pltpu.sync_copy(meta_hbm.at[1, b], meta_v.at[1])
    scrow = meta_v[0, :]
    srow = meta_v[1, :].astype(jnp.int32)
    starts = [srow[m] for m in range(M)]
    shifts = [lax.bitwise_and(starts[m], 7) for m in range(M)]
    bases = [starts[m] - shifts[m] for m in range(M)]
    svecs = [jnp.full((L,), scrow[m], jnp.float32) for m in range(M)]

    def in_copies(k, sl):
        t0 = t_base + k * C
        cps = [
            pltpu.make_async_copy(inp_hbm.at[b, pl.ds(t0, C)], inp_v.at[sl],
                                  in_sem.at[sl]),
            pltpu.make_async_copy(maskf_hbm.at[b, pl.ds(t0, C)],
                                  mask_v.at[pl.ds(sl * C, C)], in_sem.at[sl]),
        ]
        for m in range(M):
            off = pl.multiple_of(bases[m] + t0, 8)
            cps.append(pltpu.make_async_copy(
                cache_hbm.at[pl.ds(off, C + 8)], win_v.at[sl, m],
                in_sem.at[sl]))
        return cps

    def out_copy(k, sl):
        t0 = t_base + k * C
        return pltpu.make_async_copy(out_v.at[sl], out_hbm.at[b, pl.ds(t0, C)],
                                     out_sem.at[sl])

    for cp in in_copies(0, 0):
        cp.start()
    for cp in in_copies(1, 1):
        cp.start()

    def chunk_body(k, carry):
        sl = lax.rem(k, 2)
        for cp in in_copies(k, sl):
            cp.wait()
        @pl.when(k >= 2)
        def _():
            out_copy(k - 2, sl).wait()

        @plsc.parallel_loop(0, C // L, 1)
        def blk_body(rg):
            mgrp = mask_v[pl.ds(sl * C + rg * L, L)]
            for j in range(L):
                r = rg * L + j
                mv = jnp.full((L,), mgrp[j], jnp.float32)
                sm = [svecs[m] * mv for m in range(M)]
                rws = [r + shifts[m] for m in range(M)]
                for g in range(GRP):
                    cols = pl.ds(g * L, L)
                    a0 = sm[0] * win_v[sl, 0, rws[0], cols]
                    a1 = sm[1] * win_v[sl, 1, rws[1], cols]
                    a2 = sm[2] * win_v[sl, 2, rws[2], cols]
                    a3 = sm[3] * win_v[sl, 3, rws[3], cols]
                    out_v[sl, r, cols] = inp_v[sl, r, cols] + (
                        (a0 + a1) + (a2 + a3))

        out_copy(k, sl).start()
        @pl.when(k + 2 < NCHUNK)
        def _():
            for cp in in_copies(k + 2, sl):
                cp.start()
        return carry

    lax.fori_loop(0, NCHUNK, chunk_body, 0)
    out_copy(NCHUNK - 2, (NCHUNK - 2) % 2).wait()
    out_copy(NCHUNK - 1, (NCHUNK - 1) % 2).wait()


@jax.jit
def kernel(input, sequence_mask, cache, start_indices, lam_u, scale_u):
    lambda_ = LAMBDA_MIN + lam_u.astype(jnp.float32) * (LAMBDA_MAX - LAMBDA_MIN)
    ms = 0.001 + scale_u.astype(jnp.float32) * (1.0 - 0.001)
    scales = ms * lambda_ / jnp.sum(ms, axis=1, keepdims=True)
    meta = jnp.zeros((2, B, L), jnp.float32)
    meta = meta.at[0, :, :M].set(scales)
    meta = meta.at[1, :, :M].set(start_indices.astype(jnp.float32))
    maskf = sequence_mask.astype(jnp.float32)

    sc_fn = pl.kernel(
        _sc_body,
        out_type=jax.ShapeDtypeStruct((B, T, F), jnp.float32),
        mesh=plsc.VectorSubcoreMesh(core_axis_name="c", subcore_axis_name="s"),
        scratch_types=[
            pltpu.VMEM((2, L), jnp.float32),
            pltpu.VMEM((2 * C,), jnp.float32),
            pltpu.VMEM((2, C, F), jnp.float32),
            pltpu.VMEM((2, M, C + 8, F), jnp.float32),
            pltpu.VMEM((2, C, F), jnp.float32),
            pltpu.SemaphoreType.DMA((2,)),
            pltpu.SemaphoreType.DMA((2,)),
        ],
    )
    return sc_fn(input.astype(jnp.float32), maskf, cache.astype(jnp.float32),
                 meta)


# speedup vs baseline: 2.2569x; 2.2569x over previous
import jax
import jax.numpy as jnp
from jax import lax
from jax.experimental import pallas as pl
from jax.experimental.pallas import tpu as pltpu
from jax.experimental.pallas import tpu_sc as plsc

B, T, F = 16, 2048, 128
M = 4
LAMBDA_MIN, LAMBDA_MAX = 0.1, 0.4

NC, NS = 2, 16
HALF = T // NC
C = 64
NCHUNK = HALF // C
GRP = F // 16
L = 16


def _sc_body(inp_hbm, maskx_hbm, cache_hbm, meta_hbm, out_hbm,
             meta_v, maskx_v, inp_v, win_v, out_v, in_sem, out_sem):
    c = lax.axis_index("c")
    s = lax.axis_index("s")
    b = s
    t_base = c * HALF

    pltpu.sync_copy(meta_hbm.at[0, b], meta_v.at[0])
    pltpu.sync_copy(meta_hbm.at[1, b], meta_v.at[1])
    scrow = meta_v[0, :]
    srow = meta_v[1, :].astype(jnp.int32)
    starts = [srow[m] for m in range(M)]
    shifts = [lax.bitwise_and(starts[m], 7) for m in range(M)]
    bases = [starts[m] - shifts[m] for m in range(M)]
    svecs = [jnp.full((L,), scrow[m], jnp.float32) for m in range(M)]

    def in_copies(k, sl):
        t0 = t_base + k * C
        cps = [
            pltpu.make_async_copy(inp_hbm.at[b, pl.ds(t0, C)], inp_v.at[sl],
                                  in_sem.at[sl]),
            pltpu.make_async_copy(maskx_hbm.at[b, pl.ds(t0, C)],
                                  maskx_v.at[sl], in_sem.at[sl]),
        ]
        for m in range(M):
            off = pl.multiple_of(bases[m] + t0, 8)
            cps.append(pltpu.make_async_copy(
                cache_hbm.at[pl.ds(off, C + 8)], win_v.at[sl, m],
                in_sem.at[sl]))
        return cps

    def out_copy(k, sl):
        t0 = t_base + k * C
        return pltpu.make_async_copy(out_v.at[sl], out_hbm.at[b, pl.ds(t0, C)],
                                     out_sem.at[sl])

    for cp in in_copies(0, 0):
        cp.start()
    for cp in in_copies(1, 1):
        cp.start()

    def chunk_body(k, carry):
        sl = lax.rem(k, 2)
        for cp in in_copies(k, sl):
            cp.wait()
        @pl.when(k >= 2)
        def _():
            out_copy(k - 2, sl).wait()

        @plsc.parallel_loop(0, C, 1, unroll=4)
        def row_body(r):
            mv = maskx_v[sl, r, :]
            sm = [svecs[m] * mv for m in range(M)]
            rws = [r + shifts[m] for m in range(M)]
            for g in range(GRP):
                cols = pl.ds(g * L, L)
                a0 = sm[0] * win_v[sl, 0, rws[0], cols]
                a1 = sm[1] * win_v[sl, 1, rws[1], cols]
                a2 = sm[2] * win_v[sl, 2, rws[2], cols]
                a3 = sm[3] * win_v[sl, 3, rws[3], cols]
                out_v[sl, r, cols] = inp_v[sl, r, cols] + ((a0 + a1) + (a2 + a3))

        out_copy(k, sl).start()
        @pl.when(k + 2 < NCHUNK)
        def _():
            for cp in in_copies(k + 2, sl):
                cp.start()
        return carry

    lax.fori_loop(0, NCHUNK, chunk_body, 0)
    out_copy(NCHUNK - 2, (NCHUNK - 2) % 2).wait()
    out_copy(NCHUNK - 1, (NCHUNK - 1) % 2).wait()


@jax.jit
def kernel(input, sequence_mask, cache, start_indices, lam_u, scale_u):
    lambda_ = LAMBDA_MIN + lam_u.astype(jnp.float32) * (LAMBDA_MAX - LAMBDA_MIN)
    ms = 0.001 + scale_u.astype(jnp.float32) * (1.0 - 0.001)
    scales = ms * lambda_ / jnp.sum(ms, axis=1, keepdims=True)
    meta = jnp.zeros((2, B, L), jnp.float32)
    meta = meta.at[0, :, :M].set(scales)
    meta = meta.at[1, :, :M].set(start_indices.astype(jnp.float32))
    maskx = jnp.broadcast_to(
        sequence_mask.astype(jnp.float32)[:, :, None], (B, T, L))

    sc_fn = pl.kernel(
        _sc_body,
        out_type=jax.ShapeDtypeStruct((B, T, F), jnp.float32),
        mesh=plsc.VectorSubcoreMesh(core_axis_name="c", subcore_axis_name="s"),
        scratch_types=[
            pltpu.VMEM((2, L), jnp.float32),
            pltpu.VMEM((2, C, L), jnp.float32),
            pltpu.VMEM((2, C, F), jnp.float32),
            pltpu.VMEM((2, M, C + 8, F), jnp.float32),
            pltpu.VMEM((2, C, F), jnp.float32),
            pltpu.SemaphoreType.DMA((2,)),
            pltpu.SemaphoreType.DMA((2,)),
        ],
    )
    return sc_fn(input.astype(jnp.float32), maskx, cache.astype(jnp.float32),
                 meta)


# speedup vs baseline: 2.2886x; 1.0141x over previous
import jax
import jax.numpy as jnp
from jax import lax
from jax.experimental import pallas as pl
from jax.experimental.pallas import tpu as pltpu
from jax.experimental.pallas import tpu_sc as plsc

B, T, F = 16, 2048, 128
M = 4
LAMBDA_MIN, LAMBDA_MAX = 0.1, 0.4

NC, NS = 2, 16
HALF = T // NC
C = 64
NCHUNK = HALF // C
GRP = F // 16
L = 16


def _sc_body(inp_hbm, maskx_hbm, cache_hbm, meta_hbm, out_hbm,
             meta_v, maskx_v, inp_v, win_v, out_v, in_sem, out_sem):
    c = lax.axis_index("c")
    s = lax.axis_index("s")
    b = s
    t_base = c * HALF

    pltpu.sync_copy(meta_hbm.at[0, b], meta_v.at[0])
    pltpu.sync_copy(meta_hbm.at[1, b], meta_v.at[1])
    scrow = meta_v[0, :]
    srow = meta_v[1, :].astype(jnp.int32)
    starts = [srow[m] for m in range(M)]
    shifts = [lax.bitwise_and(starts[m], 7) for m in range(M)]
    bases = [starts[m] - shifts[m] for m in range(M)]
    svecs = [jnp.full((L,), scrow[m], jnp.float32) for m in range(M)]

    def in_copies(k, sl):
        t0 = t_base + k * C
        cps = [
            pltpu.make_async_copy(inp_hbm.at[b, pl.ds(t0, C)], inp_v.at[sl],
                                  in_sem.at[sl]),
            pltpu.make_async_copy(maskx_hbm.at[b, pl.ds(t0, C)],
                                  maskx_v.at[sl], in_sem.at[sl]),
        ]
        for m in range(M):
            off = pl.multiple_of(bases[m] + t0, 8)
            cps.append(pltpu.make_async_copy(
                cache_hbm.at[pl.ds(off, C + 8)], win_v.at[sl, m],
                in_sem.at[sl]))
        return cps

    def out_copy(k, sl):
        t0 = t_base + k * C
        return pltpu.make_async_copy(out_v.at[sl], out_hbm.at[b, pl.ds(t0, C)],
                                     out_sem.at[sl])

    for cp in in_copies(0, 0):
        cp.start()
    for cp in in_copies(1, 1):
        cp.start()

    def chunk_body(k, carry):
        sl = lax.rem(k, 2)
        for cp in in_copies(k, sl):
            cp.wait()
        @pl.when(k >= 2)
        def _():
            out_copy(k - 2, sl).wait()

        @plsc.parallel_loop(0, C, 1, unroll=8)
        def row_body(r):
            mv = maskx_v[sl, r, :]
            sm = [svecs[m] * mv for m in range(M)]
            rws = [r + shifts[m] for m in range(M)]
            for g in range(GRP):
                cols = pl.ds(g * L, L)
                a0 = sm[0] * win_v[sl, 0, rws[0], cols]
                a1 = sm[1] * win_v[sl, 1, rws[1], cols]
                a2 = sm[2] * win_v[sl, 2, rws[2], cols]
                a3 = sm[3] * win_v[sl, 3, rws[3], cols]
                out_v[sl, r, cols] = inp_v[sl, r, cols] + ((a0 + a1) + (a2 + a3))

        out_copy(k, sl).start()
        @pl.when(k + 2 < NCHUNK)
        def _():
            for cp in in_copies(k + 2, sl):
                cp.start()
        return carry

    lax.fori_loop(0, NCHUNK, chunk_body, 0)
    out_copy(NCHUNK - 2, (NCHUNK - 2) % 2).wait()
    out_copy(NCHUNK - 1, (NCHUNK - 1) % 2).wait()


@jax.jit
def kernel(input, sequence_mask, cache, start_indices, lam_u, scale_u):
    lambda_ = LAMBDA_MIN + lam_u.astype(jnp.float32) * (LAMBDA_MAX - LAMBDA_MIN)
    ms = 0.001 + scale_u.astype(jnp.float32) * (1.0 - 0.001)
    scales = ms * lambda_ / jnp.sum(ms, axis=1, keepdims=True)
    meta = jnp.zeros((2, B, L), jnp.float32)
    meta = meta.at[0, :, :M].set(scales)
    meta = meta.at[1, :, :M].set(start_indices.astype(jnp.float32))
    maskx = jnp.broadcast_to(
        sequence_mask.astype(jnp.float32)[:, :, None], (B, T, L))

    sc_fn = pl.kernel(
        _sc_body,
        out_type=jax.ShapeDtypeStruct((B, T, F), jnp.float32),
        mesh=plsc.VectorSubcoreMesh(core_axis_name="c", subcore_axis_name="s"),
        scratch_types=[
            pltpu.VMEM((2, L), jnp.float32),
            pltpu.VMEM((2, C, L), jnp.float32),
            pltpu.VMEM((2, C, F), jnp.float32),
            pltpu.VMEM((2, M, C + 8, F), jnp.float32),
            pltpu.VMEM((2, C, F), jnp.float32),
            pltpu.SemaphoreType.DMA((2,)),
            pltpu.SemaphoreType.DMA((2,)),
        ],
    )
    return sc_fn(input.astype(jnp.float32), maskx, cache.astype(jnp.float32),
                 meta)
